# Initial kernel scaffold; baseline (speedup 1.0000x reference)
import functools

import jax
import jax.numpy as jnp
from jax import lax
from jax.experimental import pallas as pl
from jax.experimental.pallas import tpu as pltpu
from jax.experimental.pallas import tpu_sc as plsc

NN = 10000
EE = 320000
DD = 128
NC = 2
NS = 16
L = 16
NW = NC * NS
EPT = EE // NW
CH = 128
RPT = (EPT + CH - 1) // CH
EPTP = RPT * CH
ROWS = NW * RPT
NPT = NN // NS
EPS = 1e-16

_MESH = plsc.VectorSubcoreMesh(
    core_axis_name="c", subcore_axis_name="s", num_cores=NC, num_subcores=NS)


def _wid():
    return lax.axis_index("c") * NS + lax.axis_index("s")


def _rsqrt(t):
    i = plsc.bitcast(t, jnp.int32)
    i = 0x5F3759DF - (i >> 1)
    y = plsc.bitcast(i, jnp.float32)
    for _ in range(3):
        y = y * (1.5 - 0.5 * t * y * y)
    return y


@functools.partial(
    pl.kernel,
    out_type=jax.ShapeDtypeStruct((4 * NN,), jnp.float32),
    mesh=_MESH,
    scratch_types=[
        pltpu.VMEM((RPT, CH), jnp.int32),
        pltpu.VMEM((RPT, CH), jnp.int32),
        pltpu.VMEM((EPTP,), jnp.float32),
        pltpu.VMEM_SHARED((NN,), jnp.float32),
        pltpu.VMEM_SHARED((NN,), jnp.float32),
    ],
)
def _denoms(dst2, src2, attr1, zeros_n, dparts, dst_v, src_v, ex_v, din_sc,
            dout_sc):
    cid = lax.axis_index("c")
    sid = lax.axis_index("s")
    wid = _wid()
    pltpu.sync_copy(dst2.at[pl.ds(wid * RPT, RPT)], dst_v)
    pltpu.sync_copy(src2.at[pl.ds(wid * RPT, RPT)], src_v)
    pltpu.sync_copy(attr1.at[pl.ds(wid * EPTP, EPTP)], ex_v)

    @pl.when(sid == 0)
    def _zero():
        pltpu.sync_copy(zeros_n, din_sc)
        pltpu.sync_copy(zeros_n, dout_sc)

    def _exp(i, _):
        ex_v[pl.ds(i * L, L)] = jnp.exp(ex_v[pl.ds(i * L, L)])
        return 0
    lax.fori_loop(0, EPTP // L, _exp, 0)
    plsc.subcore_barrier()

    def _scatter(j, _):
        ex_row = ex_v.at[pl.ds(j * CH, CH)]
        pltpu.sync_copy(ex_row, din_sc.at[dst_v.at[j]], add=True)
        pltpu.sync_copy(ex_row, dout_sc.at[src_v.at[j]], add=True)
        return 0
    lax.fori_loop(0, RPT, _scatter, 0)
    plsc.subcore_barrier()

    @pl.when(sid == 0)
    def _out():
        pltpu.sync_copy(din_sc, dparts.at[pl.ds(cid * NN, NN)])
        pltpu.sync_copy(dout_sc, dparts.at[pl.ds((2 + cid) * NN, NN)])


@functools.partial(
    pl.kernel,
    out_type=jax.ShapeDtypeStruct((NW * EPTP,), jnp.float32),
    mesh=_MESH,
    scratch_types=[
        pltpu.VMEM((NN,), jnp.float32),
        pltpu.VMEM((NN,), jnp.float32),
        pltpu.VMEM((NN,), jnp.float32),
        pltpu.VMEM((EPTP,), jnp.int32),
        pltpu.VMEM((EPTP,), jnp.int32),
        pltpu.VMEM((EPTP,), jnp.float32),
    ],
)
def _norms(dparts, dst1, src1, attr1, norm1, din_v, dout_v, tmp_v, dst_v,
           src_v, a_v):
    wid = _wid()
    pltpu.sync_copy(dst1.at[pl.ds(wid * EPTP, EPTP)], dst_v)
    pltpu.sync_copy(src1.at[pl.ds(wid * EPTP, EPTP)], src_v)
    pltpu.sync_copy(attr1.at[pl.ds(wid * EPTP, EPTP)], a_v)
    pltpu.sync_copy(dparts.at[pl.ds(0, NN)], din_v)
    pltpu.sync_copy(dparts.at[pl.ds(NN, NN)], tmp_v)

    def _addp(i, _):
        s = pl.ds(i * L, L)
        din_v[s] = din_v[s] + tmp_v[s]
        return 0
    lax.fori_loop(0, NN // L, _addp, 0)
    pltpu.sync_copy(dparts.at[pl.ds(2 * NN, NN)], dout_v)
    pltpu.sync_copy(dparts.at[pl.ds(3 * NN, NN)], tmp_v)

    def _addp2(i, _):
        s = pl.ds(i * L, L)
        dout_v[s] = dout_v[s] + tmp_v[s]
        return 0
    lax.fori_loop(0, NN // L, _addp2, 0)

    def _norm(i, _):
        s = pl.ds(i * L, L)
        ex = jnp.exp(a_v[s])
        d1 = plsc.load_gather(din_v, [dst_v[s]]) + EPS
        d2 = plsc.load_gather(dout_v, [src_v[s]]) + EPS
        a_v[s] = ex * _rsqrt(d1 * d2)
        return 0
    lax.fori_loop(0, EPTP // L, _norm, 0)
    pltpu.sync_copy(a_v, norm1.at[pl.ds(wid * EPTP, EPTP)])


@functools.partial(
    pl.kernel,
    out_type=jax.ShapeDtypeStruct((NC * NN, DD), jnp.float32),
    mesh=_MESH,
    scratch_types=[
        pltpu.VMEM((RPT, CH), jnp.int32),
        pltpu.VMEM((RPT, CH), jnp.int32),
        pltpu.VMEM((EPTP,), jnp.float32),
        pltpu.VMEM((CH, DD), jnp.float32),
        pltpu.VMEM_SHARED((NN, DD), jnp.float32),
        pltpu.SemaphoreType.DMA,
    ],
)
def _layer(h, dst2, src2, norm1, zeros_nd, parts, dst_v, src_v, norm_v,
           rows_v, acc_sc, sem):
    cid = lax.axis_index("c")
    sid = lax.axis_index("s")
    wid = _wid()
    pltpu.sync_copy(dst2.at[pl.ds(wid * RPT, RPT)], dst_v)
    pltpu.sync_copy(src2.at[pl.ds(wid * RPT, RPT)], src_v)
    pltpu.sync_copy(norm1.at[pl.ds(wid * EPTP, EPTP)], norm_v)
    pltpu.sync_copy(zeros_nd.at[pl.ds(sid * NPT, NPT)],
                    acc_sc.at[pl.ds(sid * NPT, NPT)])
    plsc.subcore_barrier()

    def _chunk(j, _):
        pltpu.async_copy(h.at[src_v.at[j]], rows_v, sem).wait()

        def _edge(e, _):
            nsp = plsc.load_gather(norm_v, [jnp.full((L,), j * CH + e,
                                                     jnp.int32)])
            for d in range(DD // L):
                sl = pl.ds(d * L, L)
                rows_v[e, sl] = rows_v[e, sl] * nsp
            return 0
        lax.fori_loop(0, CH, _edge, 0)
        pltpu.sync_copy(rows_v, acc_sc.at[dst_v.at[j]], add=True)
        return 0
    lax.fori_loop(0, RPT, _chunk, 0)
    plsc.subcore_barrier()
    pltpu.sync_copy(acc_sc.at[pl.ds(sid * NPT, NPT)],
                    parts.at[pl.ds(cid * NN + sid * NPT, NPT)])


def _make_combine(last):
    n_out = 1 if last else 2

    @functools.partial(
        pl.kernel,
        out_type=[jax.ShapeDtypeStruct((NN, DD), jnp.float32)] * n_out,
        mesh=_MESH,
        scratch_types=[
            pltpu.VMEM((NPT // 5, DD), jnp.float32),
            pltpu.VMEM((NPT // 5, DD), jnp.float32),
        ],
    )
    def _combine(parts, acc_prev, *rest):
        if last:
            (acc_out, a_v, b_v) = rest
        else:
            (h_out, acc_out, a_v, b_v) = rest
        cid = lax.axis_index("c")
        sid = lax.axis_index("s")
        rows = NPT // 5

        def _blk(k, _):
            base = sid * NPT + k * rows
            pltpu.sync_copy(parts.at[pl.ds(base, rows)], a_v)
            pltpu.sync_copy(parts.at[pl.ds(NN + base, rows)], b_v)

            def _add(i, _):
                s = pl.ds(i * L, L)
                fa = a_v.reshape(rows * DD)
                return 0
            def _addrow(r, _):
                def _lane(i, _):
                    s = pl.ds(i * L, L)
                    a_v[r, s] = a_v[r, s] + b_v[r, s]
                    return 0
                lax.fori_loop(0, DD // L, _lane, 0)
                return 0
            lax.fori_loop(0, rows, _addrow, 0)

            @pl.when(jnp.logical_and(cid == 0, jnp.bool_(not last)))
            def _h():
                if not last:
                    pltpu.sync_copy(a_v, h_out.at[pl.ds(base, rows)])

            @pl.when(cid == (1 if not last else 0))
            def _acc():
                pltpu.sync_copy(acc_prev.at[pl.ds(base, rows)], b_v)

                def _addrow2(r, _):
                    def _lane(i, _):
                        s = pl.ds(i * L, L)
                        if last:
                            a_v[r, s] = (a_v[r, s] + b_v[r, s]) * 0.25
                        else:
                            a_v[r, s] = a_v[r, s] + b_v[r, s]
                        return 0
                    lax.fori_loop(0, DD // L, _lane, 0)
                    return 0
                lax.fori_loop(0, rows, _addrow2, 0)
                pltpu.sync_copy(a_v, acc_out.at[pl.ds(base, rows)])
            return 0
        lax.fori_loop(0, 5, _blk, 0)

    return _combine


_combine_mid = _make_combine(last=False)
_combine_last = _make_combine(last=True)


@jax.jit
def kernel(x, edge_index, edge_attrs):
    src = edge_index[0].reshape(NW, EPT)
    dst = edge_index[1].reshape(NW, EPT)
    attr = edge_attrs.reshape(NW, EPT)
    pad = EPTP - EPT
    src2 = jnp.pad(src, ((0, 0), (0, pad))).reshape(ROWS, CH)
    dst2 = jnp.pad(dst, ((0, 0), (0, pad))).reshape(ROWS, CH)
    attr1 = jnp.pad(attr, ((0, 0), (0, pad)),
                    constant_values=-100.0).reshape(NW * EPTP)
    dst1 = dst2.reshape(NW * EPTP)
    src1 = src2.reshape(NW * EPTP)
    zeros_n = jnp.zeros((NN,), jnp.float32)
    zeros_nd = jnp.zeros((NN, DD), jnp.float32)

    dparts = _denoms(dst2, src2, attr1, zeros_n)
    norm1 = _norms(dparts, dst1, src1, attr1)

    acc = x
    h = x
    for layer in range(3):
        parts = _layer(h, dst2, src2, norm1, zeros_nd)
        if layer < 2:
            h, acc = _combine_mid(parts, acc)
        else:
            (acc,) = _combine_last(parts, acc)
    return acc


# speedup vs baseline: 7.2344x; 7.2344x over previous
import functools

import jax
import jax.numpy as jnp
from jax import lax
from jax.experimental import pallas as pl
from jax.experimental.pallas import tpu as pltpu
from jax.experimental.pallas import tpu_sc as plsc

NN = 10000
EE = 320000
DD = 128
NC = 2
NS = 16
L = 16
NW = NC * NS
EPT = EE // NW
CH = 128
RPT = 80
EPTP = RPT * CH
ROWS = NW * RPT
NNP = 10240
NPT = NNP // NS
EPS = 1e-16

_MESH = plsc.VectorSubcoreMesh(
    core_axis_name="c", subcore_axis_name="s", num_cores=NC, num_subcores=NS)


def _wid():
    return lax.axis_index("c") * NS + lax.axis_index("s")


def _rsqrt(t):
    i = plsc.bitcast(t, jnp.int32)
    i = 0x5F3759DF - (i >> 1)
    y = plsc.bitcast(i, jnp.float32)
    for _ in range(3):
        y = y * (1.5 - 0.5 * t * y * y)
    return y


@functools.partial(
    pl.kernel,
    out_type=jax.ShapeDtypeStruct((4 * NNP,), jnp.float32),
    mesh=_MESH,
    compiler_params=pltpu.CompilerParams(needs_layout_passes=False),
    scratch_types=[
        pltpu.VMEM((RPT, CH), jnp.int32),
        pltpu.VMEM((RPT, CH), jnp.int32),
        pltpu.VMEM((EPTP,), jnp.float32),
        pltpu.VMEM_SHARED((NNP,), jnp.float32),
        pltpu.VMEM_SHARED((NNP,), jnp.float32),
    ],
)
def _denoms(dst2, src2, attr1, zeros_n, dparts, dst_v, src_v, ex_v, din_sc,
            dout_sc):
    cid = lax.axis_index("c")
    sid = lax.axis_index("s")
    wid = _wid()
    pltpu.sync_copy(dst2.at[pl.ds(wid * RPT, RPT)], dst_v)
    pltpu.sync_copy(src2.at[pl.ds(wid * RPT, RPT)], src_v)
    pltpu.sync_copy(attr1.at[pl.ds(wid * EPTP, EPTP)], ex_v)

    @pl.when(sid == 0)
    def _zero():
        pltpu.sync_copy(zeros_n, din_sc)
        pltpu.sync_copy(zeros_n, dout_sc)

    def _exp(i, _):
        ex_v[pl.ds(i * L, L)] = jnp.exp(ex_v[pl.ds(i * L, L)])
        return 0
    lax.fori_loop(0, EPTP // L, _exp, 0)
    plsc.subcore_barrier()

    def _scatter(j, _):
        ex_row = ex_v.at[pl.ds(j * CH, CH)]
        pltpu.sync_copy(ex_row, din_sc.at[dst_v.at[j]], add=True)
        pltpu.sync_copy(ex_row, dout_sc.at[src_v.at[j]], add=True)
        return 0
    lax.fori_loop(0, RPT, _scatter, 0)
    plsc.subcore_barrier()

    @pl.when(sid == 0)
    def _out():
        pltpu.sync_copy(din_sc, dparts.at[pl.ds(cid * NNP, NNP)])
        pltpu.sync_copy(dout_sc, dparts.at[pl.ds((2 + cid) * NNP, NNP)])


@functools.partial(
    pl.kernel,
    out_type=jax.ShapeDtypeStruct((NW * EPTP,), jnp.float32),
    mesh=_MESH,
    compiler_params=pltpu.CompilerParams(needs_layout_passes=False),
    scratch_types=[
        pltpu.VMEM((NNP,), jnp.float32),
        pltpu.VMEM((NNP,), jnp.float32),
        pltpu.VMEM((NNP,), jnp.float32),
        pltpu.VMEM((EPTP,), jnp.int32),
        pltpu.VMEM((EPTP,), jnp.int32),
        pltpu.VMEM((EPTP,), jnp.float32),
    ],
)
def _norms(dparts, dst1, src1, attr1, norm1, din_v, dout_v, tmp_v, dst_v,
           src_v, a_v):
    wid = _wid()
    pltpu.sync_copy(dst1.at[pl.ds(wid * EPTP, EPTP)], dst_v)
    pltpu.sync_copy(src1.at[pl.ds(wid * EPTP, EPTP)], src_v)
    pltpu.sync_copy(attr1.at[pl.ds(wid * EPTP, EPTP)], a_v)
    pltpu.sync_copy(dparts.at[pl.ds(0, NNP)], din_v)
    pltpu.sync_copy(dparts.at[pl.ds(NNP, NNP)], tmp_v)

    def _addp(i, _):
        s = pl.ds(i * L, L)
        din_v[s] = din_v[s] + tmp_v[s]
        return 0
    lax.fori_loop(0, NNP // L, _addp, 0)
    pltpu.sync_copy(dparts.at[pl.ds(2 * NNP, NNP)], dout_v)
    pltpu.sync_copy(dparts.at[pl.ds(3 * NNP, NNP)], tmp_v)

    def _addp2(i, _):
        s = pl.ds(i * L, L)
        dout_v[s] = dout_v[s] + tmp_v[s]
        return 0
    lax.fori_loop(0, NNP // L, _addp2, 0)

    def _norm(i, _):
        s = pl.ds(i * L, L)
        ex = jnp.exp(a_v[s])
        d1 = plsc.load_gather(din_v, [dst_v[s]]) + EPS
        d2 = plsc.load_gather(dout_v, [src_v[s]]) + EPS
        a_v[s] = ex * _rsqrt(d1 * d2)
        return 0
    lax.fori_loop(0, EPTP // L, _norm, 0)
    pltpu.sync_copy(a_v, norm1.at[pl.ds(wid * EPTP, EPTP)])


@functools.partial(
    pl.kernel,
    out_type=jax.ShapeDtypeStruct((NC * NNP, DD), jnp.float32),
    mesh=_MESH,
    compiler_params=pltpu.CompilerParams(needs_layout_passes=False),
    scratch_types=[
        pltpu.VMEM((RPT, CH), jnp.int32),
        pltpu.VMEM((RPT, CH), jnp.int32),
        pltpu.VMEM((EPTP,), jnp.float32),
        pltpu.VMEM((CH, DD), jnp.float32),
        pltpu.VMEM_SHARED((NNP, DD), jnp.float32),
        pltpu.SemaphoreType.DMA,
    ],
)
def _layer(h, dst2, src2, norm1, zeros_nd, parts, dst_v, src_v, norm_v,
           rows_v, acc_sc, sem):
    cid = lax.axis_index("c")
    sid = lax.axis_index("s")
    wid = _wid()
    pltpu.sync_copy(dst2.at[pl.ds(wid * RPT, RPT)], dst_v)
    pltpu.sync_copy(src2.at[pl.ds(wid * RPT, RPT)], src_v)
    pltpu.sync_copy(norm1.at[pl.ds(wid * EPTP, EPTP)], norm_v)
    pltpu.sync_copy(zeros_nd.at[pl.ds(sid * NPT, NPT)],
                    acc_sc.at[pl.ds(sid * NPT, NPT)])
    plsc.subcore_barrier()

    def _chunk(j, _):
        pltpu.async_copy(h.at[src_v.at[j]], rows_v, sem).wait()

        def _edge(e, _):
            nsp = plsc.load_gather(norm_v, [jnp.full((L,), j * CH + e,
                                                     jnp.int32)])
            for d in range(DD // L):
                sl = pl.ds(d * L, L)
                rows_v[e, sl] = rows_v[e, sl] * nsp
            return 0
        lax.fori_loop(0, CH, _edge, 0)
        pltpu.sync_copy(rows_v, acc_sc.at[dst_v.at[j]], add=True)
        return 0
    lax.fori_loop(0, RPT, _chunk, 0)
    plsc.subcore_barrier()
    pltpu.sync_copy(acc_sc.at[pl.ds(sid * NPT, NPT)],
                    parts.at[pl.ds(cid * NNP + sid * NPT, NPT)])


def _make_combine(last):
    n_out = 1 if last else 2

    @functools.partial(
        pl.kernel,
        out_type=[jax.ShapeDtypeStruct((NNP, DD), jnp.float32)] * n_out,
        mesh=_MESH,
        compiler_params=pltpu.CompilerParams(needs_layout_passes=False),
        scratch_types=[
            pltpu.VMEM((NPT // 5, DD), jnp.float32),
            pltpu.VMEM((NPT // 5, DD), jnp.float32),
        ],
    )
    def _combine(parts, acc_prev, *rest):
        if last:
            (acc_out, a_v, b_v) = rest
        else:
            (h_out, acc_out, a_v, b_v) = rest
        cid = lax.axis_index("c")
        sid = lax.axis_index("s")
        rows = NPT // 5

        def _blk(k, _):
            base = sid * NPT + k * rows
            pltpu.sync_copy(parts.at[pl.ds(base, rows)], a_v)
            pltpu.sync_copy(parts.at[pl.ds(NNP + base, rows)], b_v)

            def _addrow(r, _):
                def _lane(i, _):
                    s = pl.ds(i * L, L)
                    a_v[r, s] = a_v[r, s] + b_v[r, s]
                    return 0
                lax.fori_loop(0, DD // L, _lane, 0)
                return 0
            lax.fori_loop(0, rows, _addrow, 0)

            if not last:
                @pl.when(cid == 0)
                def _h():
                    pltpu.sync_copy(a_v, h_out.at[pl.ds(base, rows)])

            @pl.when(cid == (1 if not last else 0))
            def _acc():
                pltpu.sync_copy(acc_prev.at[pl.ds(base, rows)], b_v)

                def _addrow2(r, _):
                    def _lane(i, _):
                        s = pl.ds(i * L, L)
                        if last:
                            a_v[r, s] = (a_v[r, s] + b_v[r, s]) * 0.25
                        else:
                            a_v[r, s] = a_v[r, s] + b_v[r, s]
                        return 0
                    lax.fori_loop(0, DD // L, _lane, 0)
                    return 0
                lax.fori_loop(0, rows, _addrow2, 0)
                pltpu.sync_copy(a_v, acc_out.at[pl.ds(base, rows)])
            return 0
        lax.fori_loop(0, 5, _blk, 0)

    return _combine


_combine_mid = _make_combine(last=False)
_combine_last = _make_combine(last=True)


@jax.jit
def kernel(x, edge_index, edge_attrs):
    src = edge_index[0].reshape(NW, EPT)
    dst = edge_index[1].reshape(NW, EPT)
    attr = edge_attrs.reshape(NW, EPT)
    pad = EPTP - EPT
    src2 = jnp.pad(src, ((0, 0), (0, pad))).reshape(ROWS, CH)
    dst2 = jnp.pad(dst, ((0, 0), (0, pad))).reshape(ROWS, CH)
    attr1 = jnp.pad(attr, ((0, 0), (0, pad)),
                    constant_values=-100.0).reshape(NW * EPTP)
    dst1 = dst2.reshape(NW * EPTP)
    src1 = src2.reshape(NW * EPTP)
    zeros_n = jnp.zeros((NNP,), jnp.float32)
    zeros_nd = jnp.zeros((NNP, DD), jnp.float32)
    x_p = jnp.pad(x, ((0, NNP - NN), (0, 0)))

    dparts = _denoms(dst2, src2, attr1, zeros_n)
    norm1 = _norms(dparts, dst1, src1, attr1)

    acc = x_p
    h = x_p
    for layer in range(3):
        parts = _layer(h, dst2, src2, norm1, zeros_nd)
        if layer < 2:
            h, acc = _combine_mid(parts, acc)
        else:
            (acc,) = _combine_last(parts, acc)
    return acc[:NN]


# speedup vs baseline: 8.7515x; 1.2097x over previous
import functools

import jax
import jax.numpy as jnp
from jax import lax
from jax.experimental import pallas as pl
from jax.experimental.pallas import tpu as pltpu
from jax.experimental.pallas import tpu_sc as plsc

NN = 10000
EE = 320000
DD = 128
NC = 2
NS = 16
L = 16
NW = NC * NS
EPT = EE // NW
CH = 128
RPT = 80
EPTP = RPT * CH
ROWS = NW * RPT
NNP = 10240
NPT = NNP // NS
EPS = 1e-16
MW = 2 * CH

_MESH = plsc.VectorSubcoreMesh(
    core_axis_name="c", subcore_axis_name="s", num_cores=NC, num_subcores=NS)


def _wid():
    return lax.axis_index("c") * NS + lax.axis_index("s")


def _splat(v, e):
    return lax.gather(
        v, jnp.full((L, 1), e, jnp.int32),
        dimension_numbers=lax.GatherDimensionNumbers(
            offset_dims=(), collapsed_slice_dims=(0,), start_index_map=(0,)),
        slice_sizes=(1,), mode=lax.GatherScatterMode.PROMISE_IN_BOUNDS)


def _rsqrt(t):
    i = plsc.bitcast(t, jnp.int32)
    i = 0x5F3759DF - (i >> 1)
    y = plsc.bitcast(i, jnp.float32)
    for _ in range(3):
        y = y * (1.5 - 0.5 * t * y * y)
    return y


@functools.partial(
    pl.kernel,
    out_type=jax.ShapeDtypeStruct((4 * NNP,), jnp.float32),
    mesh=_MESH,
    compiler_params=pltpu.CompilerParams(needs_layout_passes=False),
    scratch_types=[
        pltpu.VMEM((RPT, CH), jnp.int32),
        pltpu.VMEM((RPT, CH), jnp.int32),
        pltpu.VMEM((EPTP,), jnp.float32),
        pltpu.VMEM_SHARED((NNP,), jnp.float32),
        pltpu.VMEM_SHARED((NNP,), jnp.float32),
    ],
)
def _denoms(dst2, src2, attr1, zeros_n, dparts, dst_v, src_v, ex_v, din_sc,
            dout_sc):
    cid = lax.axis_index("c")
    sid = lax.axis_index("s")
    wid = _wid()
    pltpu.sync_copy(dst2.at[pl.ds(wid * RPT, RPT)], dst_v)
    pltpu.sync_copy(src2.at[pl.ds(wid * RPT, RPT)], src_v)
    pltpu.sync_copy(attr1.at[pl.ds(wid * EPTP, EPTP)], ex_v)

    @pl.when(sid == 0)
    def _zero():
        pltpu.sync_copy(zeros_n, din_sc)
        pltpu.sync_copy(zeros_n, dout_sc)

    def _exp(i, _):
        ex_v[pl.ds(i * L, L)] = jnp.exp(ex_v[pl.ds(i * L, L)])
        return 0
    lax.fori_loop(0, EPTP // L, _exp, 0)
    plsc.subcore_barrier()

    def _scatter(j, _):
        ex_row = ex_v.at[pl.ds(j * CH, CH)]
        pltpu.sync_copy(ex_row, din_sc.at[dst_v.at[j]], add=True)
        pltpu.sync_copy(ex_row, dout_sc.at[src_v.at[j]], add=True)
        return 0
    lax.fori_loop(0, RPT, _scatter, 0)
    plsc.subcore_barrier()

    @pl.when(sid == 0)
    def _out():
        pltpu.sync_copy(din_sc, dparts.at[pl.ds(cid * NNP, NNP)])
        pltpu.sync_copy(dout_sc, dparts.at[pl.ds((2 + cid) * NNP, NNP)])


@functools.partial(
    pl.kernel,
    out_type=jax.ShapeDtypeStruct((NW * EPTP * 2,), jnp.int32),
    mesh=_MESH,
    compiler_params=pltpu.CompilerParams(needs_layout_passes=False),
    scratch_types=[
        pltpu.VMEM((NNP,), jnp.float32),
        pltpu.VMEM((NNP,), jnp.float32),
        pltpu.VMEM((NNP,), jnp.float32),
        pltpu.VMEM((EPTP,), jnp.int32),
        pltpu.VMEM((EPTP,), jnp.int32),
        pltpu.VMEM((EPTP,), jnp.float32),
        pltpu.VMEM((2 * EPTP,), jnp.int32),
    ],
)
def _norms(dparts, dst1, src1, attr1, meta1, din_v, dout_v, tmp_v, dst_v,
           src_v, a_v, meta_v):
    wid = _wid()
    pltpu.sync_copy(dst1.at[pl.ds(wid * EPTP, EPTP)], dst_v)
    pltpu.sync_copy(src1.at[pl.ds(wid * EPTP, EPTP)], src_v)
    pltpu.sync_copy(attr1.at[pl.ds(wid * EPTP, EPTP)], a_v)
    pltpu.sync_copy(dparts.at[pl.ds(0, NNP)], din_v)
    pltpu.sync_copy(dparts.at[pl.ds(NNP, NNP)], tmp_v)

    def _addp(i, _):
        s = pl.ds(i * L, L)
        din_v[s] = din_v[s] + tmp_v[s]
        return 0
    lax.fori_loop(0, NNP // L, _addp, 0)
    pltpu.sync_copy(dparts.at[pl.ds(2 * NNP, NNP)], dout_v)
    pltpu.sync_copy(dparts.at[pl.ds(3 * NNP, NNP)], tmp_v)

    def _addp2(i, _):
        s = pl.ds(i * L, L)
        dout_v[s] = dout_v[s] + tmp_v[s]
        return 0
    lax.fori_loop(0, NNP // L, _addp2, 0)

    def _norm(i, _):
        s = pl.ds(i * L, L)
        ex = jnp.exp(a_v[s])
        d1 = plsc.load_gather(din_v, [dst_v[s]]) + EPS
        d2 = plsc.load_gather(dout_v, [src_v[s]]) + EPS
        a_v[s] = ex * _rsqrt(d1 * d2)
        return 0
    lax.fori_loop(0, EPTP // L, _norm, 0)

    def _meta(j, _):
        for k in range(CH // L):
            meta_v[pl.ds(j * MW + k * L, L)] = (
                src_v[pl.ds(j * CH + k * L, L)])
            meta_v[pl.ds(j * MW + CH + k * L, L)] = plsc.bitcast(
                a_v[pl.ds(j * CH + k * L, L)], jnp.int32)
        return 0
    lax.fori_loop(0, RPT, _meta, 0)
    pltpu.sync_copy(meta_v, meta1.at[pl.ds(wid * EPTP * 2, EPTP * 2)])


@functools.partial(
    pl.kernel,
    out_type=jax.ShapeDtypeStruct((NC * NNP, DD), jnp.float32),
    mesh=_MESH,
    compiler_params=pltpu.CompilerParams(needs_layout_passes=False),
    scratch_types=[
        pltpu.VMEM((RPT, CH), jnp.int32),
        [pltpu.VMEM((MW,), jnp.int32) for _ in range(2)],
        [pltpu.VMEM((CH, DD), jnp.float32) for _ in range(2)],
        pltpu.VMEM_SHARED((NNP, DD), jnp.float32),
        [pltpu.SemaphoreType.DMA for _ in range(2)],
        [pltpu.SemaphoreType.DMA for _ in range(2)],
    ],
)
def _layer(h, dst2, meta1, zeros_nd, parts, dst_v, mbufs, gbufs,
           acc_sc, msems, gsems):
    cid = lax.axis_index("c")
    sid = lax.axis_index("s")
    wid = _wid()
    mbase = wid * EPTP * 2
    pltpu.sync_copy(dst2.at[pl.ds(wid * RPT, RPT)], dst_v)
    pltpu.sync_copy(zeros_nd.at[pl.ds(sid * NPT, NPT)],
                    acc_sc.at[pl.ds(sid * NPT, NPT)])
    plsc.subcore_barrier()

    for b in range(2):
        pltpu.async_copy(meta1.at[pl.ds(mbase + b * MW, MW)], mbufs[b],
                         msems[b])
    pltpu.make_async_copy(meta1.at[pl.ds(0, MW)], mbufs[0], msems[0]).wait()
    pltpu.async_copy(h.at[mbufs[0].at[pl.ds(0, CH)]], gbufs[0], gsems[0])

    def _pair(g, _):
        for b in range(2):
            j = g * 2 + b
            mb, gb = mbufs[b], gbufs[b]
            bn = 1 - b

            @pl.when(j + 1 < RPT)
            def _ahead():
                pltpu.make_async_copy(meta1.at[pl.ds(0, MW)], mbufs[bn],
                                      msems[bn]).wait()
                pltpu.async_copy(h.at[mbufs[bn].at[pl.ds(0, CH)]],
                                 gbufs[bn], gsems[bn])

            pltpu.make_async_copy(h.at[pl.ds(0, CH)], gb, gsems[b]).wait()

            def _scale16(k, _):
                n16 = plsc.bitcast(mb[pl.ds(CH + k * L, L)], jnp.float32)
                for e16 in range(L):
                    nsp = _splat(n16, e16)
                    e = k * L + e16
                    for d in range(DD // L):
                        sl = pl.ds(d * L, L)
                        gb[e, sl] = gb[e, sl] * nsp
                return 0
            lax.fori_loop(0, CH // L, _scale16, 0)
            pltpu.sync_copy(gb, acc_sc.at[dst_v.at[j]], add=True)

            @pl.when(j + 2 < RPT)
            def _refill():
                pltpu.async_copy(
                    meta1.at[pl.ds(mbase + (j + 2) * MW, MW)], mb, msems[b])
        return 0
    lax.fori_loop(0, RPT // 2, _pair, 0)
    plsc.subcore_barrier()
    pltpu.sync_copy(acc_sc.at[pl.ds(sid * NPT, NPT)],
                    parts.at[pl.ds(cid * NNP + sid * NPT, NPT)])


def _make_combine(last):
    outs = ([jax.ShapeDtypeStruct((NNP, DD), jnp.float32)] if last else
            [jax.ShapeDtypeStruct((NNP, DD), jnp.float32),
             jax.ShapeDtypeStruct((NNP, DD), jnp.float32)])

    @functools.partial(
        pl.kernel,
        out_type=outs,
        mesh=_MESH,
        compiler_params=pltpu.CompilerParams(needs_layout_passes=False),
        scratch_types=[
            pltpu.VMEM((NPT // 5, DD), jnp.float32),
            pltpu.VMEM((NPT // 5, DD), jnp.float32),
        ],
    )
    def _combine(parts, acc_prev, *rest):
        if last:
            (acc_out, a_v, b_v) = rest
        else:
            (h_out, acc_out, a_v, b_v) = rest
        cid = lax.axis_index("c")
        sid = lax.axis_index("s")
        rows = NPT // 5

        def _blk(k, _):
            base = sid * NPT + k * rows
            pltpu.sync_copy(parts.at[pl.ds(base, rows)], a_v)
            pltpu.sync_copy(parts.at[pl.ds(NNP + base, rows)], b_v)

            def _addrow(r, _):
                for d in range(DD // L):
                    sl = pl.ds(d * L, L)
                    a_v[r, sl] = a_v[r, sl] + b_v[r, sl]
                return 0
            lax.fori_loop(0, rows, _addrow, 0)

            if not last:
                @pl.when(cid == 0)
                def _h():
                    pltpu.sync_copy(a_v, h_out.at[pl.ds(base, rows)])

            @pl.when(cid == (1 if not last else 0))
            def _acc():
                pltpu.sync_copy(acc_prev.at[pl.ds(base, rows)], b_v)

                def _addrow2(r, _):
                    for d in range(DD // L):
                        sl = pl.ds(d * L, L)
                        if last:
                            a_v[r, sl] = (a_v[r, sl] + b_v[r, sl]) * 0.25
                        else:
                            a_v[r, sl] = a_v[r, sl] + b_v[r, sl]
                    return 0
                lax.fori_loop(0, rows, _addrow2, 0)
                pltpu.sync_copy(a_v, acc_out.at[pl.ds(base, rows)])
            return 0
        lax.fori_loop(0, 5, _blk, 0)

    return _combine


_combine_mid = _make_combine(last=False)
_combine_last = _make_combine(last=True)


@jax.jit
def kernel(x, edge_index, edge_attrs):
    src = edge_index[0].reshape(NW, EPT)
    dst = edge_index[1].reshape(NW, EPT)
    attr = edge_attrs.reshape(NW, EPT)
    pad = EPTP - EPT
    src2 = jnp.pad(src, ((0, 0), (0, pad))).reshape(ROWS, CH)
    dst2 = jnp.pad(dst, ((0, 0), (0, pad))).reshape(ROWS, CH)
    attr1 = jnp.pad(attr, ((0, 0), (0, pad)),
                    constant_values=-100.0).reshape(NW * EPTP)
    dst1 = dst2.reshape(NW * EPTP)
    src1 = src2.reshape(NW * EPTP)
    zeros_n = jnp.zeros((NNP,), jnp.float32)
    zeros_nd = jnp.zeros((NNP, DD), jnp.float32)
    x_pad = jnp.pad(x, ((0, NNP - NN), (0, 0)))

    dparts = _denoms(dst2, src2, attr1, zeros_n)
    meta1 = _norms(dparts, dst1, src1, attr1)

    acc = x_pad
    h = x_pad
    for layer in range(3):
        parts = _layer(h, dst2, meta1, zeros_nd)
        if layer < 2:
            h, acc = _combine_mid(parts, acc)
        else:
            (acc,) = _combine_last(parts, acc)
    return acc[:NN]


# speedup vs baseline: 9.2360x; 1.0554x over previous
import functools

import jax
import jax.numpy as jnp
from jax import lax
from jax.experimental import pallas as pl
from jax.experimental.pallas import tpu as pltpu
from jax.experimental.pallas import tpu_sc as plsc

NN = 10000
EE = 320000
DD = 128
NC = 2
NS = 16
L = 16
NW = NC * NS
EPT = EE // NW
CH = 128
RPT = 80
EPTP = RPT * CH
ROWS = NW * RPT
NNP = 10240
NPT = NNP // NS
EPS = 1e-16
MW = 2 * CH

_MESH = plsc.VectorSubcoreMesh(
    core_axis_name="c", subcore_axis_name="s", num_cores=NC, num_subcores=NS)


def _wid():
    return lax.axis_index("c") * NS + lax.axis_index("s")


def _splat(v, e):
    return lax.gather(
        v, jnp.full((L, 1), e, jnp.int32),
        dimension_numbers=lax.GatherDimensionNumbers(
            offset_dims=(), collapsed_slice_dims=(0,), start_index_map=(0,)),
        slice_sizes=(1,), mode=lax.GatherScatterMode.PROMISE_IN_BOUNDS)


def _rsqrt(t):
    i = plsc.bitcast(t, jnp.int32)
    i = 0x5F3759DF - (i >> 1)
    y = plsc.bitcast(i, jnp.float32)
    for _ in range(3):
        y = y * (1.5 - 0.5 * t * y * y)
    return y


@functools.partial(
    pl.kernel,
    out_type=jax.ShapeDtypeStruct((4 * NNP,), jnp.float32),
    mesh=_MESH,
    compiler_params=pltpu.CompilerParams(needs_layout_passes=False),
    scratch_types=[
        pltpu.VMEM((RPT, CH), jnp.int32),
        pltpu.VMEM((RPT, CH), jnp.int32),
        pltpu.VMEM((EPTP,), jnp.float32),
        pltpu.VMEM_SHARED((NNP,), jnp.float32),
        pltpu.VMEM_SHARED((NNP,), jnp.float32),
    ],
)
def _denoms(dst2, src2, attr1, zeros_n, dparts, dst_v, src_v, ex_v, din_sc,
            dout_sc):
    cid = lax.axis_index("c")
    sid = lax.axis_index("s")
    wid = _wid()
    pltpu.sync_copy(dst2.at[pl.ds(wid * RPT, RPT)], dst_v)
    pltpu.sync_copy(src2.at[pl.ds(wid * RPT, RPT)], src_v)
    pltpu.sync_copy(attr1.at[pl.ds(wid * EPTP, EPTP)], ex_v)

    @pl.when(sid == 0)
    def _zero():
        pltpu.sync_copy(zeros_n, din_sc)
        pltpu.sync_copy(zeros_n, dout_sc)

    def _exp(i, _):
        ex_v[pl.ds(i * L, L)] = jnp.exp(ex_v[pl.ds(i * L, L)])
        return 0
    lax.fori_loop(0, EPTP // L, _exp, 0)
    plsc.subcore_barrier()

    def _scatter(j, _):
        ex_row = ex_v.at[pl.ds(j * CH, CH)]
        pltpu.sync_copy(ex_row, din_sc.at[dst_v.at[j]], add=True)
        pltpu.sync_copy(ex_row, dout_sc.at[src_v.at[j]], add=True)
        return 0
    lax.fori_loop(0, RPT, _scatter, 0)
    plsc.subcore_barrier()

    @pl.when(sid == 0)
    def _out():
        pltpu.sync_copy(din_sc, dparts.at[pl.ds(cid * NNP, NNP)])
        pltpu.sync_copy(dout_sc, dparts.at[pl.ds((2 + cid) * NNP, NNP)])


@functools.partial(
    pl.kernel,
    out_type=jax.ShapeDtypeStruct((NW * EPTP * 2,), jnp.int32),
    mesh=_MESH,
    compiler_params=pltpu.CompilerParams(needs_layout_passes=False),
    scratch_types=[
        pltpu.VMEM((NNP,), jnp.float32),
        pltpu.VMEM((NNP,), jnp.float32),
        pltpu.VMEM((NNP,), jnp.float32),
        pltpu.VMEM((EPTP,), jnp.int32),
        pltpu.VMEM((EPTP,), jnp.int32),
        pltpu.VMEM((EPTP,), jnp.float32),
        pltpu.VMEM((2 * EPTP,), jnp.int32),
    ],
)
def _norms(dparts, dst1, src1, attr1, meta1, din_v, dout_v, tmp_v, dst_v,
           src_v, a_v, meta_v):
    wid = _wid()
    pltpu.sync_copy(dst1.at[pl.ds(wid * EPTP, EPTP)], dst_v)
    pltpu.sync_copy(src1.at[pl.ds(wid * EPTP, EPTP)], src_v)
    pltpu.sync_copy(attr1.at[pl.ds(wid * EPTP, EPTP)], a_v)
    pltpu.sync_copy(dparts.at[pl.ds(0, NNP)], din_v)
    pltpu.sync_copy(dparts.at[pl.ds(NNP, NNP)], tmp_v)

    def _addp(i, _):
        s = pl.ds(i * L, L)
        din_v[s] = din_v[s] + tmp_v[s]
        return 0
    lax.fori_loop(0, NNP // L, _addp, 0)
    pltpu.sync_copy(dparts.at[pl.ds(2 * NNP, NNP)], dout_v)
    pltpu.sync_copy(dparts.at[pl.ds(3 * NNP, NNP)], tmp_v)

    def _addp2(i, _):
        s = pl.ds(i * L, L)
        dout_v[s] = dout_v[s] + tmp_v[s]
        return 0
    lax.fori_loop(0, NNP // L, _addp2, 0)

    def _norm(i, _):
        s = pl.ds(i * L, L)
        ex = jnp.exp(a_v[s])
        d1 = plsc.load_gather(din_v, [dst_v[s]]) + EPS
        d2 = plsc.load_gather(dout_v, [src_v[s]]) + EPS
        a_v[s] = ex * _rsqrt(d1 * d2)
        return 0
    lax.fori_loop(0, EPTP // L, _norm, 0)

    def _meta(j, _):
        for k in range(CH // L):
            meta_v[pl.ds(j * MW + k * L, L)] = (
                src_v[pl.ds(j * CH + k * L, L)])
            meta_v[pl.ds(j * MW + CH + k * L, L)] = plsc.bitcast(
                a_v[pl.ds(j * CH + k * L, L)], jnp.int32)
        return 0
    lax.fori_loop(0, RPT, _meta, 0)
    pltpu.sync_copy(meta_v, meta1.at[pl.ds(wid * EPTP * 2, EPTP * 2)])


@functools.partial(
    pl.kernel,
    out_type=jax.ShapeDtypeStruct((NC * NNP, DD), jnp.float32),
    mesh=_MESH,
    compiler_params=pltpu.CompilerParams(needs_layout_passes=False),
    scratch_types=[
        pltpu.VMEM((RPT, CH), jnp.int32),
        [pltpu.VMEM((MW,), jnp.int32) for _ in range(2)],
        [pltpu.VMEM((CH, DD), jnp.float32) for _ in range(2)],
        pltpu.VMEM_SHARED((NNP, DD), jnp.float32),
        [pltpu.SemaphoreType.DMA for _ in range(2)],
        [pltpu.SemaphoreType.DMA for _ in range(2)],
        [pltpu.SemaphoreType.DMA for _ in range(2)],
    ],
)
def _layer(h, dst2, meta1, zeros_nd, parts, dst_v, mbufs, gbufs,
           acc_sc, msems, gsems, ssems):
    cid = lax.axis_index("c")
    sid = lax.axis_index("s")
    wid = _wid()
    mbase = wid * EPTP * 2
    pltpu.sync_copy(dst2.at[pl.ds(wid * RPT, RPT)], dst_v)
    pltpu.sync_copy(zeros_nd.at[pl.ds(sid * NPT, NPT)],
                    acc_sc.at[pl.ds(sid * NPT, NPT)])
    plsc.subcore_barrier()

    for b in range(2):
        pltpu.async_copy(meta1.at[pl.ds(mbase + b * MW, MW)], mbufs[b],
                         msems[b])
    pltpu.make_async_copy(meta1.at[pl.ds(0, MW)], mbufs[0], msems[0]).wait()
    pltpu.async_copy(h.at[mbufs[0].at[pl.ds(0, CH)]], gbufs[0], gsems[0])

    def _pair(g, _):
        for b in range(2):
            j = g * 2 + b
            mb, gb = mbufs[b], gbufs[b]
            bn = 1 - b

            @pl.when(j >= 1)
            def _wscat():
                pltpu.make_async_copy(h.at[pl.ds(0, CH)], gbufs[bn],
                                      ssems[bn]).wait()

            @pl.when(j + 1 < RPT)
            def _ahead():
                pltpu.make_async_copy(meta1.at[pl.ds(0, MW)], mbufs[bn],
                                      msems[bn]).wait()
                pltpu.async_copy(h.at[mbufs[bn].at[pl.ds(0, CH)]],
                                 gbufs[bn], gsems[bn])

            pltpu.make_async_copy(h.at[pl.ds(0, CH)], gb, gsems[b]).wait()

            def _scale16(k, _):
                n16 = plsc.bitcast(mb[pl.ds(CH + k * L, L)], jnp.float32)
                for e16 in range(L):
                    nsp = _splat(n16, e16)
                    e = k * L + e16
                    for d in range(DD // L):
                        sl = pl.ds(d * L, L)
                        gb[e, sl] = gb[e, sl] * nsp
                return 0
            lax.fori_loop(0, CH // L, _scale16, 0)
            pltpu.async_copy(gb, acc_sc.at[dst_v.at[j]], ssems[b],
                             add=True)

            @pl.when(j + 2 < RPT)
            def _refill():
                pltpu.async_copy(
                    meta1.at[pl.ds(mbase + (j + 2) * MW, MW)], mb, msems[b])
        return 0
    lax.fori_loop(0, RPT // 2, _pair, 0)
    pltpu.make_async_copy(h.at[pl.ds(0, CH)], gbufs[(RPT - 1) % 2],
                          ssems[(RPT - 1) % 2]).wait()
    plsc.subcore_barrier()
    pltpu.sync_copy(acc_sc.at[pl.ds(sid * NPT, NPT)],
                    parts.at[pl.ds(cid * NNP + sid * NPT, NPT)])


def _make_combine(last):
    outs = ([jax.ShapeDtypeStruct((NNP, DD), jnp.float32)] if last else
            [jax.ShapeDtypeStruct((NNP, DD), jnp.float32),
             jax.ShapeDtypeStruct((NNP, DD), jnp.float32)])

    @functools.partial(
        pl.kernel,
        out_type=outs,
        mesh=_MESH,
        compiler_params=pltpu.CompilerParams(needs_layout_passes=False),
        scratch_types=[
            pltpu.VMEM((NPT // 5, DD), jnp.float32),
            pltpu.VMEM((NPT // 5, DD), jnp.float32),
        ],
    )
    def _combine(parts, acc_prev, *rest):
        if last:
            (acc_out, a_v, b_v) = rest
        else:
            (h_out, acc_out, a_v, b_v) = rest
        cid = lax.axis_index("c")
        sid = lax.axis_index("s")
        rows = NPT // 5

        def _blk(k, _):
            base = sid * NPT + k * rows
            pltpu.sync_copy(parts.at[pl.ds(base, rows)], a_v)
            pltpu.sync_copy(parts.at[pl.ds(NNP + base, rows)], b_v)

            def _addrow(r, _):
                for d in range(DD // L):
                    sl = pl.ds(d * L, L)
                    a_v[r, sl] = a_v[r, sl] + b_v[r, sl]
                return 0
            lax.fori_loop(0, rows, _addrow, 0)

            if not last:
                @pl.when(cid == 0)
                def _h():
                    pltpu.sync_copy(a_v, h_out.at[pl.ds(base, rows)])

            @pl.when(cid == (1 if not last else 0))
            def _acc():
                pltpu.sync_copy(acc_prev.at[pl.ds(base, rows)], b_v)

                def _addrow2(r, _):
                    for d in range(DD // L):
                        sl = pl.ds(d * L, L)
                        if last:
                            a_v[r, sl] = (a_v[r, sl] + b_v[r, sl]) * 0.25
                        else:
                            a_v[r, sl] = a_v[r, sl] + b_v[r, sl]
                    return 0
                lax.fori_loop(0, rows, _addrow2, 0)
                pltpu.sync_copy(a_v, acc_out.at[pl.ds(base, rows)])
            return 0
        lax.fori_loop(0, 5, _blk, 0)

    return _combine


_combine_mid = _make_combine(last=False)
_combine_last = _make_combine(last=True)


@jax.jit
def kernel(x, edge_index, edge_attrs):
    src = edge_index[0].reshape(NW, EPT)
    dst = edge_index[1].reshape(NW, EPT)
    attr = edge_attrs.reshape(NW, EPT)
    pad = EPTP - EPT
    src2 = jnp.pad(src, ((0, 0), (0, pad))).reshape(ROWS, CH)
    dst2 = jnp.pad(dst, ((0, 0), (0, pad))).reshape(ROWS, CH)
    attr1 = jnp.pad(attr, ((0, 0), (0, pad)),
                    constant_values=-100.0).reshape(NW * EPTP)
    dst1 = dst2.reshape(NW * EPTP)
    src1 = src2.reshape(NW * EPTP)
    zeros_n = jnp.zeros((NNP,), jnp.float32)
    zeros_nd = jnp.zeros((NNP, DD), jnp.float32)
    x_pad = jnp.pad(x, ((0, NNP - NN), (0, 0)))

    dparts = _denoms(dst2, src2, attr1, zeros_n)
    meta1 = _norms(dparts, dst1, src1, attr1)

    acc = x_pad
    h = x_pad
    for layer in range(3):
        parts = _layer(h, dst2, meta1, zeros_nd)
        if layer < 2:
            h, acc = _combine_mid(parts, acc)
        else:
            (acc,) = _combine_last(parts, acc)
    return acc[:NN]
